# baseline (speedup 1.0000x reference)
import functools

import jax
import jax.numpy as jnp
from jax import lax
from jax.experimental import pallas as pl
from jax.experimental.pallas import tpu as pltpu
from jax.experimental.pallas import tpu_sc as plsc

N = 10000
E = 320000
D = 128
DH = D // 2

NC = 2
NS = 16

EPT = E // NS
CHUNK = 80
NCHUNK = EPT // CHUNK
PF = 4
NBUF = 2 * PF

NPAD = 10240
SEG = 624
TAIL = N - NS * SEG

_mesh = plsc.VectorSubcoreMesh(core_axis_name="c", subcore_axis_name="s")


@functools.partial(
    pl.kernel,
    out_type=jax.ShapeDtypeStruct((NC, NPAD), jnp.float32),
    mesh=_mesh,
    scratch_types=[
        pltpu.VMEM((NCHUNK // 2, CHUNK), jnp.int32),
        pltpu.VMEM((NPAD // NS,), jnp.float32),
        pltpu.VMEM_SHARED((NPAD,), jnp.float32),
    ],
)
def _deg_kernel(dst3_hbm, deg_out_hbm, dst_v, ones_v, acc):
    c = lax.axis_index("c")
    s = lax.axis_index("s")
    wid = c * NS + s

    pltpu.sync_copy(dst3_hbm.at[wid], dst_v)

    seg = NPAD // NS
    for k in range(seg // 16):
        ones_v[pl.ds(k * 16, 16)] = jnp.full((16,), 1.0, jnp.float32)
    pltpu.sync_copy(ones_v, acc.at[pl.ds(s * seg, seg)])
    plsc.subcore_barrier()

    def body(j, carry):
        pltpu.sync_copy(ones_v.at[pl.ds(0, CHUNK)], acc.at[dst_v.at[j]],
                        add=True)
        return carry

    lax.fori_loop(0, NCHUNK // 2, body, 0)
    plsc.subcore_barrier()

    pltpu.sync_copy(acc.at[pl.ds(s * seg, seg)],
                    deg_out_hbm.at[c].at[pl.ds(s * seg, seg)])


def _mm_body(x_ref, w_ref, y_ref):
    xw = jnp.dot(x_ref[...], w_ref[...], preferred_element_type=jnp.float32)
    y_ref[0] = xw[:, :DH]
    y_ref[1] = xw[:, DH:]


def _matmul(x, W):
    blk = 1000
    return pl.pallas_call(
        _mm_body,
        grid=(N // blk,),
        in_specs=[
            pl.BlockSpec((blk, D), lambda i: (i, 0)),
            pl.BlockSpec((D, D), lambda i: (0, 0)),
        ],
        out_specs=pl.BlockSpec((NC, blk, DH), lambda i: (0, i, 0)),
        out_shape=jax.ShapeDtypeStruct((NC, N, DH), jnp.float32),
    )(x, W)


def _scale_body(xw_ref, dp_ref, y_ref):
    deg = dp_ref[0] + dp_ref[1] - 1.0
    dis = lax.rsqrt(deg)
    y_ref[0] = xw_ref[0] * dis
    y_ref[1] = xw_ref[1] * dis


def _scale(xw2, deg_cols):
    blk = 1000
    return pl.pallas_call(
        _scale_body,
        grid=(N // blk,),
        in_specs=[
            pl.BlockSpec((NC, blk, DH), lambda i: (0, i, 0)),
            pl.BlockSpec((NC, blk, 1), lambda i: (0, i, 0)),
        ],
        out_specs=pl.BlockSpec((NC, blk, DH), lambda i: (0, i, 0)),
        out_shape=jax.ShapeDtypeStruct((NC, N, DH), jnp.float32),
    )(xw2, deg_cols)


@functools.partial(
    pl.kernel,
    out_type=jax.ShapeDtypeStruct((NC, N, DH), jnp.float32),
    mesh=_mesh,
    scratch_types=[
        pltpu.VMEM((NCHUNK, CHUNK), jnp.int32),
        pltpu.VMEM((NCHUNK, CHUNK), jnp.int32),
        [pltpu.VMEM((CHUNK, DH), jnp.float32) for _ in range(NBUF)],
        [pltpu.SemaphoreType.DMA for _ in range(NBUF)],
        [pltpu.SemaphoreType.DMA for _ in range(NBUF)],
        pltpu.VMEM_SHARED((N, DH), jnp.float32),
    ],
    compiler_params=pltpu.CompilerParams(use_tc_tiling_on_sc=False,
                                         skip_device_barrier=True),
)
def _agg_kernel(y2_hbm, src3_hbm, dst3_hbm, out_hbm,
                src_v, dst_v, bufs, gsems, ssems, acc):
    c = lax.axis_index("c")
    s = lax.axis_index("s")
    yh = y2_hbm.at[c]

    pltpu.sync_copy(src3_hbm.at[s], src_v)
    pltpu.sync_copy(dst3_hbm.at[s], dst_v)

    pltpu.sync_copy(yh.at[pl.ds(s * SEG, SEG)], acc.at[pl.ds(s * SEG, SEG)])

    @pl.when(s == NS - 1)
    def _():
        pltpu.sync_copy(yh.at[pl.ds(NS * SEG, TAIL)],
                        acc.at[pl.ds(NS * SEG, TAIL)])

    plsc.subcore_barrier()

    def g(j, t):
        pltpu.async_copy(yh.at[src_v.at[j]], bufs[t], gsems[t])

    def wg(j, t):
        pltpu.make_async_copy(yh.at[src_v.at[j]], bufs[t], gsems[t]).wait()

    def sca(j, t):
        pltpu.async_copy(bufs[t], acc.at[dst_v.at[j]], ssems[t], add=True)

    def wsc(j, t):
        pltpu.make_async_copy(bufs[t], acc.at[dst_v.at[j]], ssems[t]).wait()

    for k in range(PF):
        g(k, k)
    for j in range(PF):
        wg(j, j % NBUF)
        sca(j, j % NBUF)
        g(j + PF, (j + PF) % NBUF)

    n_steady = ((NCHUNK - 2 * PF) // NBUF) * NBUF

    def steady(i, carry):
        jb = PF + NBUF * i
        for u in range(NBUF):
            j = jb + u
            t = (PF + u) % NBUF
            t2 = (t + PF) % NBUF
            wg(j, t)
            sca(j, t)
            wsc(j - PF, t2)
            g(j + PF, t2)
        return carry

    lax.fori_loop(0, n_steady // NBUF, steady, 0)

    for j in range(PF + n_steady, NCHUNK - PF):
        t = j % NBUF
        t2 = (t + PF) % NBUF
        wg(j, t)
        sca(j, t)
        wsc(j - PF, t2)
        g(j + PF, t2)
    for j in range(NCHUNK - PF, NCHUNK):
        wg(j, j % NBUF)
        sca(j, j % NBUF)
    for j in range(NCHUNK - NBUF, NCHUNK):
        wsc(j, j % NBUF)

    plsc.subcore_barrier()
    pltpu.sync_copy(acc.at[pl.ds(s * SEG, SEG)],
                    out_hbm.at[c].at[pl.ds(s * SEG, SEG)])

    @pl.when(s == NS - 1)
    def _():
        pltpu.sync_copy(acc.at[pl.ds(NS * SEG, TAIL)],
                        out_hbm.at[c].at[pl.ds(NS * SEG, TAIL)])


def _comb_body(p_ref, dp_ref, b_ref, o_ref):
    deg = dp_ref[0] + dp_ref[1] - 1.0
    dis = lax.rsqrt(deg)
    agg = jnp.concatenate([p_ref[0], p_ref[1]], axis=1)
    o_ref[...] = agg * dis + b_ref[...]


def _combine(parts, deg_cols, b):
    blk = 1000
    return pl.pallas_call(
        _comb_body,
        grid=(N // blk,),
        in_specs=[
            pl.BlockSpec((NC, blk, DH), lambda i: (0, i, 0)),
            pl.BlockSpec((NC, blk, 1), lambda i: (0, i, 0)),
            pl.BlockSpec((1, D), lambda i: (0, 0)),
        ],
        out_specs=pl.BlockSpec((blk, D), lambda i: (i, 0)),
        out_shape=jax.ShapeDtypeStruct((N, D), jnp.float32),
    )(parts, deg_cols, b.reshape(1, D))


def kernel(x, edge_index, W, b):
    src3 = edge_index[0].astype(jnp.int32).reshape(NS, NCHUNK, CHUNK)
    dst3 = edge_index[1].astype(jnp.int32).reshape(NS, NCHUNK, CHUNK)
    dst3h = dst3.reshape(NC * NS, NCHUNK // 2, CHUNK)

    xw2 = _matmul(x, W)
    deg_parts = _deg_kernel(dst3h)
    deg_cols = deg_parts.reshape(NC, NPAD, 1)
    y2 = _scale(xw2, deg_cols)
    parts = _agg_kernel(y2, src3, dst3)
    out = _combine(parts, deg_cols, b)
    return (out, 0)


# speedup vs baseline: 1.0763x; 1.0763x over previous
import functools

import jax
import jax.numpy as jnp
from jax import lax
from jax.experimental import pallas as pl
from jax.experimental.pallas import tpu as pltpu
from jax.experimental.pallas import tpu_sc as plsc

N = 10000
E = 320000
D = 128
DH = D // 2

NC = 2
NS = 16

EPT = E // NS
CHUNK = 80
NCHUNK = EPT // CHUNK
PF = 6
SD = 2
NBUF = PF + SD

NPAD = 10240
SEG = 624
TAIL = N - NS * SEG

_mesh = plsc.VectorSubcoreMesh(core_axis_name="c", subcore_axis_name="s")


@functools.partial(
    pl.kernel,
    out_type=jax.ShapeDtypeStruct((NC, NPAD), jnp.float32),
    mesh=_mesh,
    scratch_types=[
        pltpu.VMEM((NCHUNK // 2, CHUNK), jnp.int32),
        pltpu.VMEM((NPAD // NS,), jnp.float32),
        pltpu.VMEM_SHARED((NPAD,), jnp.float32),
    ],
)
def _deg_kernel(dst3_hbm, deg_out_hbm, dst_v, ones_v, acc):
    c = lax.axis_index("c")
    s = lax.axis_index("s")
    wid = c * NS + s

    pltpu.sync_copy(dst3_hbm.at[wid], dst_v)

    seg = NPAD // NS
    for k in range(seg // 16):
        ones_v[pl.ds(k * 16, 16)] = jnp.full((16,), 1.0, jnp.float32)
    pltpu.sync_copy(ones_v, acc.at[pl.ds(s * seg, seg)])
    plsc.subcore_barrier()

    def body(j, carry):
        pltpu.sync_copy(ones_v.at[pl.ds(0, CHUNK)], acc.at[dst_v.at[j]],
                        add=True)
        return carry

    lax.fori_loop(0, NCHUNK // 2, body, 0)
    plsc.subcore_barrier()

    pltpu.sync_copy(acc.at[pl.ds(s * seg, seg)],
                    deg_out_hbm.at[c].at[pl.ds(s * seg, seg)])


def _mm_body(x_ref, w_ref, dp_ref, y_ref):
    deg = dp_ref[0] + dp_ref[1] - 1.0
    dis = lax.rsqrt(deg)
    xw = jnp.dot(x_ref[...], w_ref[...], preferred_element_type=jnp.float32)
    y = xw * dis
    y_ref[0] = y[:, :DH]
    y_ref[1] = y[:, DH:]


def _matmul_scale(x, W, deg_cols):
    blk = 1000
    return pl.pallas_call(
        _mm_body,
        grid=(N // blk,),
        in_specs=[
            pl.BlockSpec((blk, D), lambda i: (i, 0)),
            pl.BlockSpec((D, D), lambda i: (0, 0)),
            pl.BlockSpec((NC, blk, 1), lambda i: (0, i, 0)),
        ],
        out_specs=pl.BlockSpec((NC, blk, DH), lambda i: (0, i, 0)),
        out_shape=jax.ShapeDtypeStruct((NC, N, DH), jnp.float32),
    )(x, W, deg_cols)


@functools.partial(
    pl.kernel,
    out_type=jax.ShapeDtypeStruct((NC, N, DH), jnp.float32),
    mesh=_mesh,
    scratch_types=[
        pltpu.VMEM((NCHUNK, CHUNK), jnp.int32),
        pltpu.VMEM((NCHUNK, CHUNK), jnp.int32),
        [pltpu.VMEM((CHUNK, DH), jnp.float32) for _ in range(NBUF)],
        [pltpu.SemaphoreType.DMA for _ in range(NBUF)],
        [pltpu.SemaphoreType.DMA for _ in range(NBUF)],
        pltpu.VMEM_SHARED((N, DH), jnp.float32),
    ],
    compiler_params=pltpu.CompilerParams(use_tc_tiling_on_sc=False),
)
def _agg_kernel(y2_hbm, src3_hbm, dst3_hbm, out_hbm,
                src_v, dst_v, bufs, gsems, ssems, acc):
    c = lax.axis_index("c")
    s = lax.axis_index("s")
    yh = y2_hbm.at[c]

    pltpu.sync_copy(src3_hbm.at[s], src_v)
    pltpu.sync_copy(dst3_hbm.at[s], dst_v)

    pltpu.sync_copy(yh.at[pl.ds(s * SEG, SEG)], acc.at[pl.ds(s * SEG, SEG)])

    @pl.when(s == NS - 1)
    def _():
        pltpu.sync_copy(yh.at[pl.ds(NS * SEG, TAIL)],
                        acc.at[pl.ds(NS * SEG, TAIL)])

    plsc.subcore_barrier()

    def g(j, t):
        pltpu.async_copy(yh.at[src_v.at[j]], bufs[t], gsems[t])

    def wg(j, t):
        pltpu.make_async_copy(yh.at[src_v.at[j]], bufs[t], gsems[t]).wait()

    def sca(j, t):
        pltpu.async_copy(bufs[t], acc.at[dst_v.at[j]], ssems[t], add=True)

    def wsc(j, t):
        pltpu.make_async_copy(bufs[t], acc.at[dst_v.at[j]], ssems[t]).wait()

    for k in range(PF):
        g(k, k % NBUF)
    for j in range(SD):
        wg(j, j % NBUF)
        sca(j, j % NBUF)
        g(j + PF, (j + PF) % NBUF)

    n_steady = ((NCHUNK - PF - SD) // NBUF) * NBUF

    def steady(i, carry):
        jb = SD + NBUF * i
        for u in range(NBUF):
            j = jb + u
            t = (SD + u) % NBUF
            t2 = (t + PF) % NBUF
            wg(j, t)
            sca(j, t)
            wsc(j - SD, t2)
            g(j + PF, t2)
        return carry

    lax.fori_loop(0, n_steady // NBUF, steady, 0)

    for j in range(SD + n_steady, NCHUNK - PF):
        t = j % NBUF
        t2 = (t + PF) % NBUF
        wg(j, t)
        sca(j, t)
        wsc(j - SD, t2)
        g(j + PF, t2)
    for j in range(NCHUNK - PF, NCHUNK):
        wg(j, j % NBUF)
        sca(j, j % NBUF)
    for j in range(NCHUNK - NBUF, NCHUNK):
        wsc(j, j % NBUF)

    plsc.subcore_barrier()
    pltpu.sync_copy(acc.at[pl.ds(s * SEG, SEG)],
                    out_hbm.at[c].at[pl.ds(s * SEG, SEG)])

    @pl.when(s == NS - 1)
    def _():
        pltpu.sync_copy(acc.at[pl.ds(NS * SEG, TAIL)],
                        out_hbm.at[c].at[pl.ds(NS * SEG, TAIL)])


def _comb_body(p_ref, dp_ref, b_ref, o_ref):
    deg = dp_ref[0] + dp_ref[1] - 1.0
    dis = lax.rsqrt(deg)
    agg = jnp.concatenate([p_ref[0], p_ref[1]], axis=1)
    o_ref[...] = agg * dis + b_ref[...]


def _combine(parts, deg_cols, b):
    blk = 1000
    return pl.pallas_call(
        _comb_body,
        grid=(N // blk,),
        in_specs=[
            pl.BlockSpec((NC, blk, DH), lambda i: (0, i, 0)),
            pl.BlockSpec((NC, blk, 1), lambda i: (0, i, 0)),
            pl.BlockSpec((1, D), lambda i: (0, 0)),
        ],
        out_specs=pl.BlockSpec((blk, D), lambda i: (i, 0)),
        out_shape=jax.ShapeDtypeStruct((N, D), jnp.float32),
    )(parts, deg_cols, b.reshape(1, D))


def kernel(x, edge_index, W, b):
    src3 = edge_index[0].astype(jnp.int32).reshape(NS, NCHUNK, CHUNK)
    dst3 = edge_index[1].astype(jnp.int32).reshape(NS, NCHUNK, CHUNK)
    dst3h = dst3.reshape(NC * NS, NCHUNK // 2, CHUNK)

    deg_parts = _deg_kernel(dst3h)
    deg_cols = deg_parts.reshape(NC, NPAD, 1)
    y2 = _matmul_scale(x, W, deg_cols)
    parts = _agg_kernel(y2, src3, dst3)
    out = _combine(parts, deg_cols, b)
    return (out, 0)


# speedup vs baseline: 1.0763x; 1.0000x over previous
import functools

import jax
import jax.numpy as jnp
from jax import lax
from jax.experimental import pallas as pl
from jax.experimental.pallas import tpu as pltpu
from jax.experimental.pallas import tpu_sc as plsc

N = 10000
E = 320000
D = 128
DH = D // 2

NC = 2
NS = 16

EPT = E // NS
CHUNK = 80
NCHUNK = EPT // CHUNK
PF = 7
SD = 2
NBUF = PF + SD

NPAD = 10240
SEG = 624
TAIL = N - NS * SEG

_mesh = plsc.VectorSubcoreMesh(core_axis_name="c", subcore_axis_name="s")


@functools.partial(
    pl.kernel,
    out_type=jax.ShapeDtypeStruct((NC, NPAD), jnp.float32),
    mesh=_mesh,
    scratch_types=[
        pltpu.VMEM((NCHUNK // 2, CHUNK), jnp.int32),
        pltpu.VMEM((NPAD // NS,), jnp.float32),
        pltpu.VMEM_SHARED((NPAD,), jnp.float32),
    ],
)
def _deg_kernel(dst3_hbm, deg_out_hbm, dst_v, ones_v, acc):
    c = lax.axis_index("c")
    s = lax.axis_index("s")
    wid = c * NS + s

    pltpu.sync_copy(dst3_hbm.at[wid], dst_v)

    seg = NPAD // NS
    for k in range(seg // 16):
        ones_v[pl.ds(k * 16, 16)] = jnp.full((16,), 1.0, jnp.float32)
    pltpu.sync_copy(ones_v, acc.at[pl.ds(s * seg, seg)])
    plsc.subcore_barrier()

    def body(j, carry):
        pltpu.sync_copy(ones_v.at[pl.ds(0, CHUNK)], acc.at[dst_v.at[j]],
                        add=True)
        return carry

    lax.fori_loop(0, NCHUNK // 2, body, 0)
    plsc.subcore_barrier()

    pltpu.sync_copy(acc.at[pl.ds(s * seg, seg)],
                    deg_out_hbm.at[c].at[pl.ds(s * seg, seg)])


def _mm_body(x_ref, w_ref, dp_ref, y_ref):
    deg = dp_ref[0] + dp_ref[1] - 1.0
    dis = lax.rsqrt(deg)
    xw = jnp.dot(x_ref[...], w_ref[...], preferred_element_type=jnp.float32)
    y = xw * dis
    y_ref[0] = y[:, :DH]
    y_ref[1] = y[:, DH:]


def _matmul_scale(x, W, deg_cols):
    blk = 1000
    return pl.pallas_call(
        _mm_body,
        grid=(N // blk,),
        in_specs=[
            pl.BlockSpec((blk, D), lambda i: (i, 0)),
            pl.BlockSpec((D, D), lambda i: (0, 0)),
            pl.BlockSpec((NC, blk, 1), lambda i: (0, i, 0)),
        ],
        out_specs=pl.BlockSpec((NC, blk, DH), lambda i: (0, i, 0)),
        out_shape=jax.ShapeDtypeStruct((NC, N, DH), jnp.float32),
    )(x, W, deg_cols)


@functools.partial(
    pl.kernel,
    out_type=jax.ShapeDtypeStruct((NC, N, DH), jnp.float32),
    mesh=_mesh,
    scratch_types=[
        pltpu.VMEM((NCHUNK, CHUNK), jnp.int32),
        pltpu.VMEM((NCHUNK, CHUNK), jnp.int32),
        [pltpu.VMEM((CHUNK, DH), jnp.float32) for _ in range(NBUF)],
        [pltpu.SemaphoreType.DMA for _ in range(NBUF)],
        [pltpu.SemaphoreType.DMA for _ in range(NBUF)],
        pltpu.VMEM_SHARED((N, DH), jnp.float32),
    ],
    compiler_params=pltpu.CompilerParams(use_tc_tiling_on_sc=False),
)
def _agg_kernel(y2_hbm, src3_hbm, dst3_hbm, out_hbm,
                src_v, dst_v, bufs, gsems, ssems, acc):
    c = lax.axis_index("c")
    s = lax.axis_index("s")
    yh = y2_hbm.at[c]

    pltpu.sync_copy(src3_hbm.at[s], src_v)
    pltpu.sync_copy(dst3_hbm.at[s], dst_v)

    pltpu.sync_copy(yh.at[pl.ds(s * SEG, SEG)], acc.at[pl.ds(s * SEG, SEG)])

    @pl.when(s == NS - 1)
    def _():
        pltpu.sync_copy(yh.at[pl.ds(NS * SEG, TAIL)],
                        acc.at[pl.ds(NS * SEG, TAIL)])

    plsc.subcore_barrier()

    def g(j, t):
        pltpu.async_copy(yh.at[src_v.at[j]], bufs[t], gsems[t])

    def wg(j, t):
        pltpu.make_async_copy(yh.at[src_v.at[j]], bufs[t], gsems[t]).wait()

    def sca(j, t):
        pltpu.async_copy(bufs[t], acc.at[dst_v.at[j]], ssems[t], add=True)

    def wsc(j, t):
        pltpu.make_async_copy(bufs[t], acc.at[dst_v.at[j]], ssems[t]).wait()

    for k in range(PF):
        g(k, k % NBUF)
    for j in range(SD):
        wg(j, j % NBUF)
        sca(j, j % NBUF)
        g(j + PF, (j + PF) % NBUF)

    n_steady = ((NCHUNK - PF - SD) // NBUF) * NBUF

    def steady(i, carry):
        jb = SD + NBUF * i
        for u in range(NBUF):
            j = jb + u
            t = (SD + u) % NBUF
            t2 = (t + PF) % NBUF
            wg(j, t)
            sca(j, t)
            wsc(j - SD, t2)
            g(j + PF, t2)
        return carry

    lax.fori_loop(0, n_steady // NBUF, steady, 0)

    for j in range(SD + n_steady, NCHUNK - PF):
        t = j % NBUF
        t2 = (t + PF) % NBUF
        wg(j, t)
        sca(j, t)
        wsc(j - SD, t2)
        g(j + PF, t2)
    for j in range(NCHUNK - PF, NCHUNK):
        wg(j, j % NBUF)
        sca(j, j % NBUF)
    for j in range(NCHUNK - NBUF, NCHUNK):
        wsc(j, j % NBUF)

    plsc.subcore_barrier()
    pltpu.sync_copy(acc.at[pl.ds(s * SEG, SEG)],
                    out_hbm.at[c].at[pl.ds(s * SEG, SEG)])

    @pl.when(s == NS - 1)
    def _():
        pltpu.sync_copy(acc.at[pl.ds(NS * SEG, TAIL)],
                        out_hbm.at[c].at[pl.ds(NS * SEG, TAIL)])


def _comb_body(p_ref, dp_ref, b_ref, o_ref):
    deg = dp_ref[0] + dp_ref[1] - 1.0
    dis = lax.rsqrt(deg)
    agg = jnp.concatenate([p_ref[0], p_ref[1]], axis=1)
    o_ref[...] = agg * dis + b_ref[...]


def _combine(parts, deg_cols, b):
    blk = 1000
    return pl.pallas_call(
        _comb_body,
        grid=(N // blk,),
        in_specs=[
            pl.BlockSpec((NC, blk, DH), lambda i: (0, i, 0)),
            pl.BlockSpec((NC, blk, 1), lambda i: (0, i, 0)),
            pl.BlockSpec((1, D), lambda i: (0, 0)),
        ],
        out_specs=pl.BlockSpec((blk, D), lambda i: (i, 0)),
        out_shape=jax.ShapeDtypeStruct((N, D), jnp.float32),
    )(parts, deg_cols, b.reshape(1, D))


def kernel(x, edge_index, W, b):
    src3 = edge_index[0].astype(jnp.int32).reshape(NS, NCHUNK, CHUNK)
    dst3 = edge_index[1].astype(jnp.int32).reshape(NS, NCHUNK, CHUNK)
    dst3h = dst3.reshape(NC * NS, NCHUNK // 2, CHUNK)

    deg_parts = _deg_kernel(dst3h)
    deg_cols = deg_parts.reshape(NC, NPAD, 1)
    y2 = _matmul_scale(x, W, deg_cols)
    parts = _agg_kernel(y2, src3, dst3)
    out = _combine(parts, deg_cols, b)
    return (out, 0)


# speedup vs baseline: 1.0961x; 1.0184x over previous
import functools

import jax
import jax.numpy as jnp
from jax import lax
from jax.experimental import pallas as pl
from jax.experimental.pallas import tpu as pltpu
from jax.experimental.pallas import tpu_sc as plsc

N = 10000
E = 320000
D = 128
DH = D // 2

NC = 2
NS = 16

EPT = E // NS
CHUNK = 80
NCHUNK = EPT // CHUNK
PF = 6
SD = 2
NBUF = PF + SD

NPAD = 10240
SEG = 624
TAIL = N - NS * SEG

_mesh = plsc.VectorSubcoreMesh(core_axis_name="c", subcore_axis_name="s")


@functools.partial(
    pl.kernel,
    out_type=jax.ShapeDtypeStruct((NC, NPAD), jnp.float32),
    mesh=_mesh,
    scratch_types=[
        pltpu.VMEM((NCHUNK // 2, CHUNK), jnp.int32),
        pltpu.VMEM((NPAD // NS,), jnp.float32),
        pltpu.VMEM_SHARED((NPAD,), jnp.float32),
    ],
)
def _deg_kernel(dst3_hbm, deg_out_hbm, dst_v, ones_v, acc):
    c = lax.axis_index("c")
    s = lax.axis_index("s")
    wid = c * NS + s

    pltpu.sync_copy(dst3_hbm.at[wid], dst_v)

    seg = NPAD // NS
    for k in range(seg // 16):
        ones_v[pl.ds(k * 16, 16)] = jnp.full((16,), 1.0, jnp.float32)
    pltpu.sync_copy(ones_v, acc.at[pl.ds(s * seg, seg)])
    plsc.subcore_barrier()

    def body(j, carry):
        pltpu.sync_copy(ones_v.at[pl.ds(0, CHUNK)], acc.at[dst_v.at[j]],
                        add=True)
        return carry

    lax.fori_loop(0, NCHUNK // 2, body, 0)
    plsc.subcore_barrier()

    pltpu.sync_copy(acc.at[pl.ds(s * seg, seg)],
                    deg_out_hbm.at[c].at[pl.ds(s * seg, seg)])


def _mm_body(x_ref, w_ref, dp_ref, y_ref):
    deg = dp_ref[0] + dp_ref[1] - 1.0
    dis = lax.rsqrt(deg)
    xw = jnp.dot(x_ref[...], w_ref[...], preferred_element_type=jnp.float32)
    y = xw * dis
    y_ref[0] = y[:, :DH]
    y_ref[1] = y[:, DH:]


def _matmul_scale(x, W, deg_cols):
    blk = 2000
    return pl.pallas_call(
        _mm_body,
        grid=(N // blk,),
        in_specs=[
            pl.BlockSpec((blk, D), lambda i: (i, 0)),
            pl.BlockSpec((D, D), lambda i: (0, 0)),
            pl.BlockSpec((NC, blk, 1), lambda i: (0, i, 0)),
        ],
        out_specs=pl.BlockSpec((NC, blk, DH), lambda i: (0, i, 0)),
        out_shape=jax.ShapeDtypeStruct((NC, N, DH), jnp.float32),
    )(x, W, deg_cols)


@functools.partial(
    pl.kernel,
    out_type=jax.ShapeDtypeStruct((NC, N, DH), jnp.float32),
    mesh=_mesh,
    scratch_types=[
        pltpu.VMEM((NCHUNK, CHUNK), jnp.int32),
        pltpu.VMEM((NCHUNK, CHUNK), jnp.int32),
        [pltpu.VMEM((CHUNK, DH), jnp.float32) for _ in range(NBUF)],
        [pltpu.SemaphoreType.DMA for _ in range(NBUF)],
        [pltpu.SemaphoreType.DMA for _ in range(NBUF)],
        pltpu.VMEM_SHARED((N, DH), jnp.float32),
    ],
    compiler_params=pltpu.CompilerParams(use_tc_tiling_on_sc=False),
)
def _agg_kernel(y2_hbm, src3_hbm, dst3_hbm, out_hbm,
                src_v, dst_v, bufs, gsems, ssems, acc):
    c = lax.axis_index("c")
    s = lax.axis_index("s")
    yh = y2_hbm.at[c]

    pltpu.sync_copy(src3_hbm.at[s], src_v)
    pltpu.sync_copy(dst3_hbm.at[s], dst_v)

    pltpu.sync_copy(yh.at[pl.ds(s * SEG, SEG)], acc.at[pl.ds(s * SEG, SEG)])

    @pl.when(s == NS - 1)
    def _():
        pltpu.sync_copy(yh.at[pl.ds(NS * SEG, TAIL)],
                        acc.at[pl.ds(NS * SEG, TAIL)])

    plsc.subcore_barrier()

    def g(j, t):
        pltpu.async_copy(yh.at[src_v.at[j]], bufs[t], gsems[t])

    def wg(j, t):
        pltpu.make_async_copy(yh.at[src_v.at[j]], bufs[t], gsems[t]).wait()

    def sca(j, t):
        pltpu.async_copy(bufs[t], acc.at[dst_v.at[j]], ssems[t], add=True)

    def wsc(j, t):
        pltpu.make_async_copy(bufs[t], acc.at[dst_v.at[j]], ssems[t]).wait()

    for k in range(PF):
        g(k, k % NBUF)
    for j in range(SD):
        wg(j, j % NBUF)
        sca(j, j % NBUF)
        g(j + PF, (j + PF) % NBUF)

    n_steady = ((NCHUNK - PF - SD) // NBUF) * NBUF

    def steady(i, carry):
        jb = SD + NBUF * i
        for u in range(NBUF):
            j = jb + u
            t = (SD + u) % NBUF
            t2 = (t + PF) % NBUF
            wg(j, t)
            sca(j, t)
            wsc(j - SD, t2)
            g(j + PF, t2)
        return carry

    lax.fori_loop(0, n_steady // NBUF, steady, 0)

    for j in range(SD + n_steady, NCHUNK - PF):
        t = j % NBUF
        t2 = (t + PF) % NBUF
        wg(j, t)
        sca(j, t)
        wsc(j - SD, t2)
        g(j + PF, t2)
    for j in range(NCHUNK - PF, NCHUNK):
        wg(j, j % NBUF)
        sca(j, j % NBUF)
    for j in range(NCHUNK - NBUF, NCHUNK):
        wsc(j, j % NBUF)

    plsc.subcore_barrier()
    pltpu.sync_copy(acc.at[pl.ds(s * SEG, SEG)],
                    out_hbm.at[c].at[pl.ds(s * SEG, SEG)])

    @pl.when(s == NS - 1)
    def _():
        pltpu.sync_copy(acc.at[pl.ds(NS * SEG, TAIL)],
                        out_hbm.at[c].at[pl.ds(NS * SEG, TAIL)])


def _comb_body(p_ref, dp_ref, b_ref, o_ref):
    deg = dp_ref[0] + dp_ref[1] - 1.0
    dis = lax.rsqrt(deg)
    agg = jnp.concatenate([p_ref[0], p_ref[1]], axis=1)
    o_ref[...] = agg * dis + b_ref[...]


def _combine(parts, deg_cols, b):
    blk = 2000
    return pl.pallas_call(
        _comb_body,
        grid=(N // blk,),
        in_specs=[
            pl.BlockSpec((NC, blk, DH), lambda i: (0, i, 0)),
            pl.BlockSpec((NC, blk, 1), lambda i: (0, i, 0)),
            pl.BlockSpec((1, D), lambda i: (0, 0)),
        ],
        out_specs=pl.BlockSpec((blk, D), lambda i: (i, 0)),
        out_shape=jax.ShapeDtypeStruct((N, D), jnp.float32),
    )(parts, deg_cols, b.reshape(1, D))


def kernel(x, edge_index, W, b):
    src3 = edge_index[0].astype(jnp.int32).reshape(NS, NCHUNK, CHUNK)
    dst3 = edge_index[1].astype(jnp.int32).reshape(NS, NCHUNK, CHUNK)
    dst3h = dst3.reshape(NC * NS, NCHUNK // 2, CHUNK)

    deg_parts = _deg_kernel(dst3h)
    deg_cols = deg_parts.reshape(NC, NPAD, 1)
    y2 = _matmul_scale(x, W, deg_cols)
    parts = _agg_kernel(y2, src3, dst3)
    out = _combine(parts, deg_cols, b)
    return (out, 0)


# speedup vs baseline: 1.1469x; 1.0464x over previous
import functools

import jax
import jax.numpy as jnp
from jax import lax
from jax.experimental import pallas as pl
from jax.experimental.pallas import tpu as pltpu
from jax.experimental.pallas import tpu_sc as plsc

N = 10000
E = 320000
D = 128
DH = D // 2

NC = 2
NS = 16

EPT = E // NS
CHUNK = 80
NCHUNK = EPT // CHUNK
PF = 6
SD = 2
NBUF = PF + SD

NPAD = 10240
SEG = 624
TAIL = N - NS * SEG

_mesh = plsc.VectorSubcoreMesh(core_axis_name="c", subcore_axis_name="s")


@functools.partial(
    pl.kernel,
    out_type=jax.ShapeDtypeStruct((NC, NPAD), jnp.float32),
    mesh=_mesh,
    scratch_types=[
        pltpu.VMEM((NCHUNK // 2, CHUNK), jnp.int32),
        pltpu.VMEM((NPAD // NS,), jnp.float32),
        pltpu.VMEM_SHARED((NPAD,), jnp.float32),
        [pltpu.SemaphoreType.DMA for _ in range(8)],
    ],
)
def _deg_kernel(dst3_hbm, deg_out_hbm, dst_v, ones_v, acc, dsems):
    c = lax.axis_index("c")
    s = lax.axis_index("s")
    wid = c * NS + s

    pltpu.sync_copy(dst3_hbm.at[wid], dst_v)

    seg = NPAD // NS
    for k in range(seg // 16):
        ones_v[pl.ds(k * 16, 16)] = jnp.full((16,), 1.0, jnp.float32)
    pltpu.sync_copy(ones_v, acc.at[pl.ds(s * seg, seg)])
    plsc.subcore_barrier()

    NW = NCHUNK // 2

    def issue(j, t):
        pltpu.async_copy(ones_v.at[pl.ds(0, CHUNK)], acc.at[dst_v.at[j]],
                         dsems[t], add=True)

    def drain(j, t):
        pltpu.make_async_copy(ones_v.at[pl.ds(0, CHUNK)],
                              acc.at[dst_v.at[j]], dsems[t]).wait()

    for k in range(8):
        issue(k, k)

    def body(i, carry):
        jb = 8 + 8 * i
        for u in range(8):
            drain(jb + u - 8, u)
            issue(jb + u, u)
        return carry

    n_mid = ((NW - 8) // 8) * 8
    lax.fori_loop(0, n_mid // 8, body, 0)
    for j in range(8 + n_mid, NW):
        drain(j - 8, j % 8)
        issue(j, j % 8)
    for j in range(NW - 8, NW):
        drain(j, j % 8)
    plsc.subcore_barrier()

    pltpu.sync_copy(acc.at[pl.ds(s * seg, seg)],
                    deg_out_hbm.at[c].at[pl.ds(s * seg, seg)])


def _mm_body(x_ref, w_ref, dp_ref, y_ref):
    deg = dp_ref[0] + dp_ref[1] - 1.0
    dis = lax.rsqrt(deg)
    xw = jnp.dot(x_ref[...], w_ref[...], preferred_element_type=jnp.float32)
    y = xw * dis
    y_ref[0] = y[:, :DH]
    y_ref[1] = y[:, DH:]


def _matmul_scale(x, W, deg_cols):
    blk = 2000
    return pl.pallas_call(
        _mm_body,
        grid=(N // blk,),
        in_specs=[
            pl.BlockSpec((blk, D), lambda i: (i, 0)),
            pl.BlockSpec((D, D), lambda i: (0, 0)),
            pl.BlockSpec((NC, blk, 1), lambda i: (0, i, 0)),
        ],
        out_specs=pl.BlockSpec((NC, blk, DH), lambda i: (0, i, 0)),
        out_shape=jax.ShapeDtypeStruct((NC, N, DH), jnp.float32),
    )(x, W, deg_cols)


@functools.partial(
    pl.kernel,
    out_type=jax.ShapeDtypeStruct((NC, N, DH), jnp.float32),
    mesh=_mesh,
    scratch_types=[
        pltpu.VMEM((NCHUNK, CHUNK), jnp.int32),
        pltpu.VMEM((NCHUNK, CHUNK), jnp.int32),
        [pltpu.VMEM((CHUNK, DH), jnp.float32) for _ in range(NBUF)],
        [pltpu.SemaphoreType.DMA for _ in range(NBUF)],
        [pltpu.SemaphoreType.DMA for _ in range(NBUF)],
        pltpu.VMEM_SHARED((N, DH), jnp.float32),
    ],
    compiler_params=pltpu.CompilerParams(use_tc_tiling_on_sc=False),
)
def _agg_kernel(y2_hbm, src3_hbm, dst3_hbm, out_hbm,
                src_v, dst_v, bufs, gsems, ssems, acc):
    c = lax.axis_index("c")
    s = lax.axis_index("s")
    yh = y2_hbm.at[c]

    pltpu.sync_copy(src3_hbm.at[s], src_v)
    pltpu.sync_copy(dst3_hbm.at[s], dst_v)

    pltpu.sync_copy(yh.at[pl.ds(s * SEG, SEG)], acc.at[pl.ds(s * SEG, SEG)])

    @pl.when(s == NS - 1)
    def _():
        pltpu.sync_copy(yh.at[pl.ds(NS * SEG, TAIL)],
                        acc.at[pl.ds(NS * SEG, TAIL)])

    plsc.subcore_barrier()

    def g(j, t):
        pltpu.async_copy(yh.at[src_v.at[j]], bufs[t], gsems[t])

    def wg(j, t):
        pltpu.make_async_copy(yh.at[src_v.at[j]], bufs[t], gsems[t]).wait()

    def sca(j, t):
        pltpu.async_copy(bufs[t], acc.at[dst_v.at[j]], ssems[t], add=True)

    def wsc(j, t):
        pltpu.make_async_copy(bufs[t], acc.at[dst_v.at[j]], ssems[t]).wait()

    for k in range(PF):
        g(k, k % NBUF)
    for j in range(SD):
        wg(j, j % NBUF)
        sca(j, j % NBUF)
        g(j + PF, (j + PF) % NBUF)

    n_steady = ((NCHUNK - PF - SD) // NBUF) * NBUF

    def steady(i, carry):
        jb = SD + NBUF * i
        for u in range(NBUF):
            j = jb + u
            t = (SD + u) % NBUF
            t2 = (t + PF) % NBUF
            wg(j, t)
            sca(j, t)
            wsc(j - SD, t2)
            g(j + PF, t2)
        return carry

    lax.fori_loop(0, n_steady // NBUF, steady, 0)

    for j in range(SD + n_steady, NCHUNK - PF):
        t = j % NBUF
        t2 = (t + PF) % NBUF
        wg(j, t)
        sca(j, t)
        wsc(j - SD, t2)
        g(j + PF, t2)
    for j in range(NCHUNK - PF, NCHUNK):
        wg(j, j % NBUF)
        sca(j, j % NBUF)
    for j in range(NCHUNK - NBUF, NCHUNK):
        wsc(j, j % NBUF)

    plsc.subcore_barrier()
    pltpu.sync_copy(acc.at[pl.ds(s * SEG, SEG)],
                    out_hbm.at[c].at[pl.ds(s * SEG, SEG)])

    @pl.when(s == NS - 1)
    def _():
        pltpu.sync_copy(acc.at[pl.ds(NS * SEG, TAIL)],
                        out_hbm.at[c].at[pl.ds(NS * SEG, TAIL)])


def _comb_body(p_ref, dp_ref, b_ref, o_ref):
    deg = dp_ref[0] + dp_ref[1] - 1.0
    dis = lax.rsqrt(deg)
    agg = jnp.concatenate([p_ref[0], p_ref[1]], axis=1)
    o_ref[...] = agg * dis + b_ref[...]


def _combine(parts, deg_cols, b):
    blk = 2000
    return pl.pallas_call(
        _comb_body,
        grid=(N // blk,),
        in_specs=[
            pl.BlockSpec((NC, blk, DH), lambda i: (0, i, 0)),
            pl.BlockSpec((NC, blk, 1), lambda i: (0, i, 0)),
            pl.BlockSpec((1, D), lambda i: (0, 0)),
        ],
        out_specs=pl.BlockSpec((blk, D), lambda i: (i, 0)),
        out_shape=jax.ShapeDtypeStruct((N, D), jnp.float32),
    )(parts, deg_cols, b.reshape(1, D))


def kernel(x, edge_index, W, b):
    src3 = edge_index[0].astype(jnp.int32).reshape(NS, NCHUNK, CHUNK)
    dst3 = edge_index[1].astype(jnp.int32).reshape(NS, NCHUNK, CHUNK)
    dst3h = dst3.reshape(NC * NS, NCHUNK // 2, CHUNK)

    deg_parts = _deg_kernel(dst3h)
    deg_cols = deg_parts.reshape(NC, NPAD, 1)
    y2 = _matmul_scale(x, W, deg_cols)
    parts = _agg_kernel(y2, src3, dst3)
    out = _combine(parts, deg_cols, b)
    return (out, 0)


# speedup vs baseline: 1.1661x; 1.0167x over previous
import functools

import jax
import jax.numpy as jnp
from jax import lax
from jax.experimental import pallas as pl
from jax.experimental.pallas import tpu as pltpu
from jax.experimental.pallas import tpu_sc as plsc

N = 10000
E = 320000
D = 128
DH = D // 2

NC = 2
NS = 16

EPT = E // NS
CHUNK = 80
NCHUNK = EPT // CHUNK
PF = 6
SD = 2
NBUF = PF + SD

NPAD = 10240
SEG = 624
TAIL = N - NS * SEG

_mesh = plsc.VectorSubcoreMesh(core_axis_name="c", subcore_axis_name="s")


@functools.partial(
    pl.kernel,
    out_type=jax.ShapeDtypeStruct((NC, NPAD), jnp.float32),
    mesh=_mesh,
    scratch_types=[
        pltpu.VMEM((NCHUNK // 2, CHUNK), jnp.int32),
        pltpu.VMEM((NPAD // NS,), jnp.float32),
        pltpu.VMEM_SHARED((NPAD,), jnp.float32),
        [pltpu.SemaphoreType.DMA for _ in range(8)],
    ],
)
def _deg_kernel(dst3_hbm, deg_out_hbm, dst_v, ones_v, acc, dsems):
    c = lax.axis_index("c")
    s = lax.axis_index("s")
    wid = c * NS + s

    pltpu.sync_copy(dst3_hbm.at[wid], dst_v)

    seg = NPAD // NS
    for k in range(seg // 16):
        ones_v[pl.ds(k * 16, 16)] = jnp.full((16,), 1.0, jnp.float32)
    pltpu.sync_copy(ones_v, acc.at[pl.ds(s * seg, seg)])
    plsc.subcore_barrier()

    NW = NCHUNK // 2

    def issue(j, t):
        pltpu.async_copy(ones_v.at[pl.ds(0, CHUNK)], acc.at[dst_v.at[j]],
                         dsems[t], add=True)

    def drain(j, t):
        pltpu.make_async_copy(ones_v.at[pl.ds(0, CHUNK)],
                              acc.at[dst_v.at[j]], dsems[t]).wait()

    for k in range(8):
        issue(k, k)

    def body(i, carry):
        jb = 8 + 8 * i
        for u in range(8):
            drain(jb + u - 8, u)
            issue(jb + u, u)
        return carry

    n_mid = ((NW - 8) // 8) * 8
    lax.fori_loop(0, n_mid // 8, body, 0)
    for j in range(8 + n_mid, NW):
        drain(j - 8, j % 8)
        issue(j, j % 8)
    for j in range(NW - 8, NW):
        drain(j, j % 8)
    plsc.subcore_barrier()

    pltpu.sync_copy(acc.at[pl.ds(s * seg, seg)],
                    deg_out_hbm.at[c].at[pl.ds(s * seg, seg)])


def _mm_body(x_ref, w_ref, dp_ref, y_ref):
    deg = dp_ref[0] + dp_ref[1] - 1.0
    dis = lax.rsqrt(deg)
    xw = jnp.dot(x_ref[...], w_ref[...], preferred_element_type=jnp.float32)
    y = xw * dis
    y_ref[0] = y[:, :DH]
    y_ref[1] = y[:, DH:]


def _matmul_scale(x, W, deg_cols):
    blk = 2000
    return pl.pallas_call(
        _mm_body,
        grid=(N // blk,),
        in_specs=[
            pl.BlockSpec((blk, D), lambda i: (i, 0)),
            pl.BlockSpec((D, D), lambda i: (0, 0)),
            pl.BlockSpec((NC, blk, 1), lambda i: (0, i, 0)),
        ],
        out_specs=pl.BlockSpec((NC, blk, DH), lambda i: (0, i, 0)),
        out_shape=jax.ShapeDtypeStruct((NC, N, DH), jnp.float32),
    )(x, W, deg_cols)


@functools.partial(
    pl.kernel,
    out_type=jax.ShapeDtypeStruct((NC, N, DH), jnp.float32),
    mesh=_mesh,
    scratch_types=[
        pltpu.VMEM((NCHUNK, CHUNK), jnp.int32),
        pltpu.VMEM((NCHUNK, CHUNK), jnp.int32),
        [pltpu.VMEM((CHUNK, DH), jnp.float32) for _ in range(NBUF)],
        [pltpu.SemaphoreType.DMA for _ in range(NBUF)],
        [pltpu.SemaphoreType.DMA for _ in range(NBUF)],
        pltpu.VMEM_SHARED((N, DH), jnp.float32),
    ],
    compiler_params=pltpu.CompilerParams(use_tc_tiling_on_sc=False),
)
def _agg_kernel(y2_hbm, src3_hbm, dst3_hbm, out_hbm,
                src_v, dst_v, bufs, gsems, ssems, acc):
    c = lax.axis_index("c")
    s = lax.axis_index("s")
    yh = y2_hbm.at[c]

    c_idx0 = pltpu.async_copy(src3_hbm.at[s], src_v, gsems[0])
    c_idx1 = pltpu.async_copy(dst3_hbm.at[s], dst_v, gsems[1])
    c_init = pltpu.async_copy(yh.at[pl.ds(s * SEG, SEG)],
                              acc.at[pl.ds(s * SEG, SEG)], gsems[2])

    @pl.when(s == NS - 1)
    def _():
        pltpu.sync_copy(yh.at[pl.ds(NS * SEG, TAIL)],
                        acc.at[pl.ds(NS * SEG, TAIL)])

    c_idx0.wait()
    c_idx1.wait()
    c_init.wait()
    plsc.subcore_barrier()

    def g(j, t):
        pltpu.async_copy(yh.at[src_v.at[j]], bufs[t], gsems[t])

    def wg(j, t):
        pltpu.make_async_copy(yh.at[src_v.at[j]], bufs[t], gsems[t]).wait()

    def sca(j, t):
        pltpu.async_copy(bufs[t], acc.at[dst_v.at[j]], ssems[t], add=True)

    def wsc(j, t):
        pltpu.make_async_copy(bufs[t], acc.at[dst_v.at[j]], ssems[t]).wait()

    for k in range(PF):
        g(k, k % NBUF)
    for j in range(SD):
        wg(j, j % NBUF)
        sca(j, j % NBUF)
        g(j + PF, (j + PF) % NBUF)

    n_steady = ((NCHUNK - PF - SD) // NBUF) * NBUF

    def steady(i, carry):
        jb = SD + NBUF * i
        for u in range(NBUF):
            j = jb + u
            t = (SD + u) % NBUF
            t2 = (t + PF) % NBUF
            wg(j, t)
            sca(j, t)
            wsc(j - SD, t2)
            g(j + PF, t2)
        return carry

    lax.fori_loop(0, n_steady // NBUF, steady, 0)

    for j in range(SD + n_steady, NCHUNK - PF):
        t = j % NBUF
        t2 = (t + PF) % NBUF
        wg(j, t)
        sca(j, t)
        wsc(j - SD, t2)
        g(j + PF, t2)
    for j in range(NCHUNK - PF, NCHUNK):
        wg(j, j % NBUF)
        sca(j, j % NBUF)
    for j in range(NCHUNK - NBUF, NCHUNK):
        wsc(j, j % NBUF)

    plsc.subcore_barrier()
    pltpu.sync_copy(acc.at[pl.ds(s * SEG, SEG)],
                    out_hbm.at[c].at[pl.ds(s * SEG, SEG)])

    @pl.when(s == NS - 1)
    def _():
        pltpu.sync_copy(acc.at[pl.ds(NS * SEG, TAIL)],
                        out_hbm.at[c].at[pl.ds(NS * SEG, TAIL)])


def _comb_body(p_ref, dp_ref, b_ref, o_ref):
    deg = dp_ref[0] + dp_ref[1] - 1.0
    dis = lax.rsqrt(deg)
    agg = jnp.concatenate([p_ref[0], p_ref[1]], axis=1)
    o_ref[...] = agg * dis + b_ref[...]


def _combine(parts, deg_cols, b):
    blk = 2000
    return pl.pallas_call(
        _comb_body,
        grid=(N // blk,),
        in_specs=[
            pl.BlockSpec((NC, blk, DH), lambda i: (0, i, 0)),
            pl.BlockSpec((NC, blk, 1), lambda i: (0, i, 0)),
            pl.BlockSpec((1, D), lambda i: (0, 0)),
        ],
        out_specs=pl.BlockSpec((blk, D), lambda i: (i, 0)),
        out_shape=jax.ShapeDtypeStruct((N, D), jnp.float32),
    )(parts, deg_cols, b.reshape(1, D))


def kernel(x, edge_index, W, b):
    src3 = edge_index[0].astype(jnp.int32).reshape(NS, NCHUNK, CHUNK)
    dst3 = edge_index[1].astype(jnp.int32).reshape(NS, NCHUNK, CHUNK)
    dst3h = dst3.reshape(NC * NS, NCHUNK // 2, CHUNK)

    deg_parts = _deg_kernel(dst3h)
    deg_cols = deg_parts.reshape(NC, NPAD, 1)
    y2 = _matmul_scale(x, W, deg_cols)
    parts = _agg_kernel(y2, src3, dst3)
    out = _combine(parts, deg_cols, b)
    return (out, 0)
